# baseline (speedup 1.0000x reference)
import jax
import jax.numpy as jnp
from jax import lax
from jax.experimental import pallas as pl
from jax.experimental.pallas import tpu as pltpu
from jax.experimental.pallas import tpu_sc as plsc

B = 16384
EMB = 256
SRC = 256
NC = 2
NS = 16
NW = NC * NS
B_PER_W = B // NW
CH = 128
NCH = B_PER_W // CH
BM = 2048


def _sc_gather_body(table_hbm, idx_hbm, out_hbm, idx_v, fb0, fb1,
                    sem_g, sem_s0, sem_s1):
    wid = lax.axis_index("s") * NC + lax.axis_index("c")
    base = wid * B_PER_W
    pltpu.sync_copy(idx_hbm.at[pl.ds(base, B_PER_W)], idx_v)
    fbufs = (fb0, fb1)
    sems = (sem_s0, sem_s1)
    scat = [None, None]
    g = pltpu.async_copy(table_hbm.at[idx_v.at[pl.ds(0, CH)]], fb0, sem_g)
    for c in range(NCH):
        g.wait()
        if c + 1 < NCH:
            if scat[(c + 1) % 2] is not None:
                scat[(c + 1) % 2].wait()
            g = pltpu.async_copy(
                table_hbm.at[idx_v.at[pl.ds((c + 1) * CH, CH)]],
                fbufs[(c + 1) % 2], sem_g)
        scat[c % 2] = pltpu.async_copy(
            fbufs[c % 2], out_hbm.at[pl.ds(base + c * CH, CH)], sems[c % 2])
    scat[0].wait()
    scat[1].wait()


_sc_gather = pl.kernel(
    _sc_gather_body,
    out_type=jax.ShapeDtypeStruct((B, EMB), jnp.float32),
    mesh=plsc.VectorSubcoreMesh(core_axis_name="c", subcore_axis_name="s"),
    compiler_params=pltpu.CompilerParams(needs_layout_passes=False),
    scratch_types=[
        pltpu.VMEM((B_PER_W,), jnp.int32),
        pltpu.VMEM((CH, EMB), jnp.float32),
        pltpu.VMEM((CH, EMB), jnp.float32),
        pltpu.SemaphoreType.DMA,
        pltpu.SemaphoreType.DMA,
        pltpu.SemaphoreType.DMA,
    ],
)

_DN = (((1,), (1,)), ((), ()))


def _mm_body(x_ref, w0_ref, w1_ref, o_ref):
    x = x_ref[...]
    o_ref[:, 0, :] = lax.dot_general(x, w0_ref[...], _DN,
                                     preferred_element_type=jnp.float32)
    o_ref[:, 1, :] = lax.dot_general(x, w1_ref[...], _DN,
                                     preferred_element_type=jnp.float32)


_matmul = pl.pallas_call(
    _mm_body,
    grid=(B // BM,),
    in_specs=[
        pl.BlockSpec((BM, EMB), lambda i: (i, 0)),
        pl.BlockSpec((SRC, EMB), lambda i: (0, 0)),
        pl.BlockSpec((SRC, EMB), lambda i: (0, 0)),
    ],
    out_specs=pl.BlockSpec((BM, 2, SRC), lambda i: (i, 0, 0)),
    out_shape=jax.ShapeDtypeStruct((B, 2, SRC), jnp.float32),
)


@jax.jit
def _run(indexes, entity_table, W0, W1):
    emb = _sc_gather(entity_table, indexes)
    return _matmul(emb, W0, W1)


def kernel(indexes, entity_table, W0, W1):
    return _run(indexes, entity_table, W0, W1)


# speedup vs baseline: 1.0164x; 1.0164x over previous
import jax
import jax.numpy as jnp
from jax import lax
from jax.experimental import pallas as pl
from jax.experimental.pallas import tpu as pltpu
from jax.experimental.pallas import tpu_sc as plsc

B = 16384
EMB = 256
SRC = 256
NC = 2
NS = 16
NW = NC * NS
B_PER_W = B // NW
CH = 128
NCH = B_PER_W // CH
BM = 8192


def _sc_gather_body(table_hbm, idx_hbm, out_hbm, idx_v, fb0, fb1,
                    sem_g, sem_s0, sem_s1):
    wid = lax.axis_index("s") * NC + lax.axis_index("c")
    base = wid * B_PER_W
    pltpu.sync_copy(idx_hbm.at[pl.ds(base, B_PER_W)], idx_v)
    fbufs = (fb0, fb1)
    sems = (sem_s0, sem_s1)
    scat = [None, None]
    g = pltpu.async_copy(table_hbm.at[idx_v.at[pl.ds(0, CH)]], fb0, sem_g)
    for c in range(NCH):
        g.wait()
        if c + 1 < NCH:
            if scat[(c + 1) % 2] is not None:
                scat[(c + 1) % 2].wait()
            g = pltpu.async_copy(
                table_hbm.at[idx_v.at[pl.ds((c + 1) * CH, CH)]],
                fbufs[(c + 1) % 2], sem_g)
        scat[c % 2] = pltpu.async_copy(
            fbufs[c % 2], out_hbm.at[pl.ds(base + c * CH, CH)], sems[c % 2])
    scat[0].wait()
    scat[1].wait()


_sc_gather = pl.kernel(
    _sc_gather_body,
    out_type=jax.ShapeDtypeStruct((B, EMB), jnp.float32),
    mesh=plsc.VectorSubcoreMesh(core_axis_name="c", subcore_axis_name="s"),
    compiler_params=pltpu.CompilerParams(needs_layout_passes=False),
    scratch_types=[
        pltpu.VMEM((B_PER_W,), jnp.int32),
        pltpu.VMEM((CH, EMB), jnp.float32),
        pltpu.VMEM((CH, EMB), jnp.float32),
        pltpu.SemaphoreType.DMA,
        pltpu.SemaphoreType.DMA,
        pltpu.SemaphoreType.DMA,
    ],
)

_DN = (((1,), (1,)), ((), ()))


def _mm_body(x_ref, w0_ref, w1_ref, o_ref):
    x = x_ref[...]
    o_ref[:, 0, :] = lax.dot_general(x, w0_ref[...], _DN,
                                     preferred_element_type=jnp.float32)
    o_ref[:, 1, :] = lax.dot_general(x, w1_ref[...], _DN,
                                     preferred_element_type=jnp.float32)


_matmul = pl.pallas_call(
    _mm_body,
    grid=(B // BM,),
    in_specs=[
        pl.BlockSpec((BM, EMB), lambda i: (i, 0)),
        pl.BlockSpec((SRC, EMB), lambda i: (0, 0)),
        pl.BlockSpec((SRC, EMB), lambda i: (0, 0)),
    ],
    out_specs=pl.BlockSpec((BM, 2, SRC), lambda i: (i, 0, 0)),
    out_shape=jax.ShapeDtypeStruct((B, 2, SRC), jnp.float32),
)


@jax.jit
def _run(indexes, entity_table, W0, W1):
    emb = _sc_gather(entity_table, indexes)
    return _matmul(emb, W0, W1)


def kernel(indexes, entity_table, W0, W1):
    return _run(indexes, entity_table, W0, W1)


# speedup vs baseline: 1.0614x; 1.0443x over previous
import jax
import jax.numpy as jnp
from jax import lax
from jax.experimental import pallas as pl
from jax.experimental.pallas import tpu as pltpu
from jax.experimental.pallas import tpu_sc as plsc

B = 16384
EMB = 256
SRC = 256
NC = 2
NS = 16
NW = NC * NS
B_PER_W = B // NW
CH = 128
NCH = B_PER_W // CH
BM = 4096


def _sc_gather_body(table_hbm, idx_hbm, out_hbm, idx_v,
                    fb0, fb1, fb2, sg0, sg1, sg2, ss0, ss1, ss2):
    wid = lax.axis_index("s") * NC + lax.axis_index("c")
    base = wid * B_PER_W
    pltpu.sync_copy(idx_hbm.at[pl.ds(base, B_PER_W)], idx_v)
    fbufs = (fb0, fb1, fb2)
    gsems = (sg0, sg1, sg2)
    ssems = (ss0, ss1, ss2)
    DEPTH = 2
    gd = [None] * NCH
    scat = [None, None, None]
    for c in range(min(DEPTH, NCH)):
        gd[c] = pltpu.async_copy(
            table_hbm.at[idx_v.at[pl.ds(c * CH, CH)]], fbufs[c % 3],
            gsems[c % 3])
    for c in range(NCH):
        gd[c].wait()
        nxt = c + DEPTH
        if nxt < NCH:
            nb = nxt % 3
            if scat[nb] is not None:
                scat[nb].wait()
            gd[nxt] = pltpu.async_copy(
                table_hbm.at[idx_v.at[pl.ds(nxt * CH, CH)]], fbufs[nb],
                gsems[nb])
        scat[c % 3] = pltpu.async_copy(
            fbufs[c % 3], out_hbm.at[pl.ds(base + c * CH, CH)], ssems[c % 3])
    for s in scat:
        if s is not None:
            s.wait()


_sc_gather = pl.kernel(
    _sc_gather_body,
    out_type=jax.ShapeDtypeStruct((B, EMB), jnp.float32),
    mesh=plsc.VectorSubcoreMesh(core_axis_name="c", subcore_axis_name="s"),
    compiler_params=pltpu.CompilerParams(needs_layout_passes=False),
    scratch_types=[
        pltpu.VMEM((B_PER_W,), jnp.int32),
        pltpu.VMEM((CH, EMB), jnp.float32),
        pltpu.VMEM((CH, EMB), jnp.float32),
        pltpu.VMEM((CH, EMB), jnp.float32),
        pltpu.SemaphoreType.DMA,
        pltpu.SemaphoreType.DMA,
        pltpu.SemaphoreType.DMA,
        pltpu.SemaphoreType.DMA,
        pltpu.SemaphoreType.DMA,
        pltpu.SemaphoreType.DMA,
    ],
)

_DN = (((1,), (1,)), ((), ()))


def _mm_body(x_ref, w0_ref, w1_ref, o_ref):
    x = x_ref[...]
    o_ref[:, 0, :] = lax.dot_general(x, w0_ref[...], _DN,
                                     preferred_element_type=jnp.float32)
    o_ref[:, 1, :] = lax.dot_general(x, w1_ref[...], _DN,
                                     preferred_element_type=jnp.float32)


_matmul = pl.pallas_call(
    _mm_body,
    grid=(B // BM,),
    in_specs=[
        pl.BlockSpec((BM, EMB), lambda i: (i, 0)),
        pl.BlockSpec((SRC, EMB), lambda i: (0, 0)),
        pl.BlockSpec((SRC, EMB), lambda i: (0, 0)),
    ],
    out_specs=pl.BlockSpec((BM, 2, SRC), lambda i: (i, 0, 0)),
    out_shape=jax.ShapeDtypeStruct((B, 2, SRC), jnp.float32),
)


@jax.jit
def _run(indexes, entity_table, W0, W1):
    emb = _sc_gather(entity_table, indexes)
    return _matmul(emb, W0, W1)


def kernel(indexes, entity_table, W0, W1):
    return _run(indexes, entity_table, W0, W1)
